# Initial kernel scaffold; baseline (speedup 1.0000x reference)
import jax
import jax.numpy as jnp
from jax.experimental import pallas as pl


def kernel(coord, charges, edge_index, mol_idx):
    raise NotImplementedError("write your pallas kernel here")


# speedup vs baseline: 279.6358x; 279.6358x over previous
import functools

import jax
import jax.numpy as jnp
import numpy as np
from jax import lax
from jax.experimental import pallas as pl
from jax.experimental.pallas import tpu as pltpu
from jax.experimental.pallas import tpu_sc as plsc

_RC = 4.6
_FACTOR = 0.5 * 27.211386245988 * 0.529177210903
_NUM_MOLS = 100
_N_ATOMS = 100000
_N_EDGES = 3200000

_NC = 2
_NS = 16
_NW = _NC * _NS

_G = 128
_SUBS = 16
_B = _G * _SUBS
_NCH = -(-_N_EDGES // (_NW * _B))
_EPT = _NCH * _B
_E_PAD = _NW * _EPT


def _tile_body(table, iarr, jarr, out, ibuf, jbuf, rows_i, rows_j, acc, obuf,
               sem):
    wid = lax.axis_index("s") * _NC + lax.axis_index("c")
    lanes = jnp.arange(16, dtype=jnp.int32)
    zero16 = jnp.zeros(16, dtype=jnp.float32)

    for r in range(16):
        for cg in range(8):
            acc[r, pl.ds(cg * 16, 16)] = zero16

    def chunk_body(c, carry):
        cid = wid * jnp.int32(_NCH) + c
        pltpu.sync_copy(iarr.at[cid], ibuf)
        pltpu.sync_copy(jarr.at[cid], jbuf)

        def sub_body(g, carry2):
            pltpu.async_copy(table.at[ibuf.at[g]], rows_i, sem).wait()
            pltpu.async_copy(table.at[jbuf.at[g]], rows_j, sem).wait()
            for v in range(_G // 16):
                rsel = lanes + v * 16
                iv = ibuf[g, pl.ds(v * 16, 16)]
                jv = jbuf[g, pl.ds(v * 16, 16)]
                xi = plsc.load_gather(rows_i, [rsel, lanes * 0])
                yi = plsc.load_gather(rows_i, [rsel, lanes * 0 + 1])
                zi = plsc.load_gather(rows_i, [rsel, lanes * 0 + 2])
                qi = plsc.load_gather(rows_i, [rsel, lanes * 0 + 3])
                molf = plsc.load_gather(rows_i, [rsel, lanes * 0 + 4])
                xj = plsc.load_gather(rows_j, [rsel, lanes * 0])
                yj = plsc.load_gather(rows_j, [rsel, lanes * 0 + 1])
                zj = plsc.load_gather(rows_j, [rsel, lanes * 0 + 2])
                qj = plsc.load_gather(rows_j, [rsel, lanes * 0 + 3])

                dx = xi - xj
                dy = yi - yj
                dz = zi - zj
                r2 = dx * dx + dy * dy + dz * dz + np.float32(1e-12)
                seed = jnp.int32(0x5F3759DF) - (plsc.bitcast(r2, jnp.int32) >> 1)
                y = plsc.bitcast(seed, jnp.float32)
                for _ in range(3):
                    y = y * (np.float32(1.5)
                             - np.float32(0.5) * r2 * y * y)
                inv_d = y
                d = r2 * inv_d
                x = d * np.float32(1.0 / _RC)
                t = jnp.maximum(np.float32(1.0) - x * x, np.float32(1e-6))
                fc = jnp.exp(np.float32(1.0) - np.float32(1.0) / t)
                fc = jnp.where(d < np.float32(_RC), fc, np.float32(0.0))

                mol = molf.astype(jnp.int32)

                e = (np.float32(1.0) - fc) * (qi * qj) * inv_d
                e = jnp.where(iv != jv, e, np.float32(0.0))
                plsc.addupdate_scatter(acc, [lanes, mol], e)
            return carry2

        lax.fori_loop(jnp.int32(0), jnp.int32(_SUBS), sub_body,
                      jnp.int32(0), unroll=False)
        return carry

    lax.fori_loop(jnp.int32(0), jnp.int32(_NCH), chunk_body,
                  jnp.int32(0), unroll=False)

    for cg in range(8):
        s = acc[0, pl.ds(cg * 16, 16)]
        for r in range(1, 16):
            s = s + acc[r, pl.ds(cg * 16, 16)]
        obuf[pl.ds(cg * 16, 16)] = s
    pltpu.sync_copy(obuf, out.at[wid])


@jax.jit
def _lr_coulomb_sc(table, iarr, jarr):
    mesh = plsc.VectorSubcoreMesh(core_axis_name="c", subcore_axis_name="s")
    f = pl.kernel(
        _tile_body,
        out_type=jax.ShapeDtypeStruct((_NW, 128), jnp.float32),
        mesh=mesh,
        compiler_params=pltpu.CompilerParams(
            needs_layout_passes=False, use_tc_tiling_on_sc=False),
        scratch_types=[
            pltpu.VMEM((_SUBS, _G), jnp.int32),
            pltpu.VMEM((_SUBS, _G), jnp.int32),
            pltpu.VMEM((_G, 8), jnp.float32),
            pltpu.VMEM((_G, 8), jnp.float32),
            pltpu.VMEM((16, 128), jnp.float32),
            pltpu.VMEM((128,), jnp.float32),
            pltpu.SemaphoreType.DMA,
        ],
    )
    return f(table, iarr, jarr)


def kernel(coord, charges, edge_index, mol_idx):
    coord = coord.astype(jnp.float32)
    q = charges.astype(jnp.float32)
    molf = mol_idx.astype(jnp.float32)
    table = jnp.concatenate(
        [coord, q[:, None], molf[:, None],
         jnp.zeros((_N_ATOMS, 3), jnp.float32)], axis=1)

    i32 = edge_index[0].astype(jnp.int32)
    j32 = edge_index[1].astype(jnp.int32)
    pad = _E_PAD - _N_EDGES
    iarr = jnp.pad(i32, (0, pad)).reshape(_NW * _NCH, _SUBS, _G)
    jarr = jnp.pad(j32, (0, pad)).reshape(_NW * _NCH, _SUBS, _G)

    partials = _lr_coulomb_sc(table, iarr, jarr)
    e_mol = jnp.sum(partials.astype(jnp.float64), axis=0)[:_NUM_MOLS]
    return _FACTOR * e_mol


# speedup vs baseline: 602.7312x; 2.1554x over previous
import jax
import jax.numpy as jnp
import numpy as np
from jax import lax
from jax.experimental import pallas as pl
from jax.experimental.pallas import tpu as pltpu
from jax.experimental.pallas import tpu_sc as plsc

_RC = 4.6
_FACTOR = 0.5 * 27.211386245988 * 0.529177210903
_NUM_MOLS = 100
_N_ATOMS = 100000
_N_EDGES = 3200000

_NC = 2
_NS = 16
_NW = _NC * _NS

_G = 128
_SUBS = 16
_B = _G * _SUBS
_NCH = -(-_N_EDGES // (_NW * _B))
_EPT = _NCH * _B
_E_PAD = _NW * _EPT
_S = _NCH * _SUBS
_NBUF = 4


def _tile_body(table, iarr, jarr, out, ibuf, jbuf, rows_i, rows_j, acc, obuf,
               sem_rows, sem_idx):
    wid = lax.axis_index("s") * _NC + lax.axis_index("c")
    lanes = jnp.arange(16, dtype=jnp.int32)
    zero16 = jnp.zeros(16, dtype=jnp.float32)

    for r in range(16):
        for cg in range(8):
            acc[r, pl.ds(cg * 16, 16)] = zero16

    def issue_rows(s2):
        c2 = s2 >> 4
        g2 = s2 & jnp.int32(15)
        cp2 = c2 & jnp.int32(1)
        slot = s2 & jnp.int32(_NBUF - 1)
        pltpu.async_copy(table.at[ibuf.at[cp2, g2]], rows_i.at[slot], sem_rows)
        pltpu.async_copy(table.at[jbuf.at[cp2, g2]], rows_j.at[slot], sem_rows)

    pltpu.sync_copy(iarr.at[wid * jnp.int32(_NCH)], ibuf.at[jnp.int32(0)])
    pltpu.sync_copy(jarr.at[wid * jnp.int32(_NCH)], jbuf.at[jnp.int32(0)])
    for s0 in range(_NBUF - 1):
        issue_rows(jnp.int32(s0))

    def loop_body(s, carry):
        c = s >> 4
        g = s & jnp.int32(15)
        cp = c & jnp.int32(1)
        slot = s & jnp.int32(_NBUF - 1)

        s2 = s + jnp.int32(_NBUF - 1)

        @pl.when(s2 < jnp.int32(_S))
        def _():
            c2 = s2 >> 4
            g2 = s2 & jnp.int32(15)
            cp2 = c2 & jnp.int32(1)

            @pl.when((g2 == jnp.int32(0)) & (c2 > jnp.int32(0)))
            def _():
                cid2 = wid * jnp.int32(_NCH) + c2
                pltpu.make_async_copy(iarr.at[cid2], ibuf.at[cp2],
                                      sem_idx).wait()
                pltpu.make_async_copy(jarr.at[cid2], jbuf.at[cp2],
                                      sem_idx).wait()

            @pl.when((g2 == jnp.int32(4)) & (c2 + jnp.int32(1) < jnp.int32(_NCH)))
            def _():
                cid3 = wid * jnp.int32(_NCH) + c2 + jnp.int32(1)
                cp3 = cp2 ^ jnp.int32(1)
                pltpu.async_copy(iarr.at[cid3], ibuf.at[cp3], sem_idx)
                pltpu.async_copy(jarr.at[cid3], jbuf.at[cp3], sem_idx)

            issue_rows(s2)

        pltpu.make_async_copy(table.at[ibuf.at[cp, g]], rows_i.at[slot],
                              sem_rows).wait()
        pltpu.make_async_copy(table.at[jbuf.at[cp, g]], rows_j.at[slot],
                              sem_rows).wait()

        ri = rows_i.at[slot]
        rj = rows_j.at[slot]
        for v in range(_G // 16):
            rsel = lanes + jnp.int32(v * 16)
            col0 = lanes * 0
            iv = ibuf[cp, g, pl.ds(v * 16, 16)]
            jv = jbuf[cp, g, pl.ds(v * 16, 16)]
            xi = plsc.load_gather(ri, [rsel, col0])
            yi = plsc.load_gather(ri, [rsel, col0 + 1])
            zi = plsc.load_gather(ri, [rsel, col0 + 2])
            qi = plsc.load_gather(ri, [rsel, col0 + 3])
            molf = plsc.load_gather(ri, [rsel, col0 + 4])
            xj = plsc.load_gather(rj, [rsel, col0])
            yj = plsc.load_gather(rj, [rsel, col0 + 1])
            zj = plsc.load_gather(rj, [rsel, col0 + 2])
            qj = plsc.load_gather(rj, [rsel, col0 + 3])

            dx = xi - xj
            dy = yi - yj
            dz = zi - zj
            r2 = dx * dx + dy * dy + dz * dz + np.float32(1e-12)
            seed = jnp.int32(0x5F3759DF) - (plsc.bitcast(r2, jnp.int32) >> 1)
            y = plsc.bitcast(seed, jnp.float32)
            for _ in range(3):
                y = y * (np.float32(1.5) - np.float32(0.5) * r2 * y * y)
            inv_d = y
            d = r2 * inv_d
            x = d * np.float32(1.0 / _RC)
            t = jnp.maximum(np.float32(1.0) - x * x, np.float32(1e-6))
            fc = jnp.exp(np.float32(1.0) - np.float32(1.0) / t)
            fc = jnp.where(d < np.float32(_RC), fc, np.float32(0.0))

            mol = molf.astype(jnp.int32)
            e = (np.float32(1.0) - fc) * (qi * qj) * inv_d
            e = jnp.where(iv != jv, e, np.float32(0.0))
            plsc.addupdate_scatter(acc, [lanes, mol], e)
        return carry

    lax.fori_loop(jnp.int32(0), jnp.int32(_S), loop_body, jnp.int32(0),
                  unroll=False)

    for cg in range(8):
        s = acc[0, pl.ds(cg * 16, 16)]
        for r in range(1, 16):
            s = s + acc[r, pl.ds(cg * 16, 16)]
        obuf[pl.ds(cg * 16, 16)] = s
    pltpu.sync_copy(obuf, out.at[wid])


@jax.jit
def _lr_coulomb_sc(table, iarr, jarr):
    mesh = plsc.VectorSubcoreMesh(core_axis_name="c", subcore_axis_name="s")
    f = pl.kernel(
        _tile_body,
        out_type=jax.ShapeDtypeStruct((_NW, 128), jnp.float32),
        mesh=mesh,
        compiler_params=pltpu.CompilerParams(
            needs_layout_passes=False, use_tc_tiling_on_sc=False),
        scratch_types=[
            pltpu.VMEM((2, _SUBS, _G), jnp.int32),
            pltpu.VMEM((2, _SUBS, _G), jnp.int32),
            pltpu.VMEM((_NBUF, _G, 8), jnp.float32),
            pltpu.VMEM((_NBUF, _G, 8), jnp.float32),
            pltpu.VMEM((16, 128), jnp.float32),
            pltpu.VMEM((128,), jnp.float32),
            pltpu.SemaphoreType.DMA,
            pltpu.SemaphoreType.DMA,
        ],
    )
    return f(table, iarr, jarr)


def kernel(coord, charges, edge_index, mol_idx):
    coord = coord.astype(jnp.float32)
    q = charges.astype(jnp.float32)
    molf = mol_idx.astype(jnp.float32)
    table = jnp.concatenate(
        [coord, q[:, None], molf[:, None],
         jnp.zeros((_N_ATOMS, 3), jnp.float32)], axis=1)

    i32 = edge_index[0].astype(jnp.int32)
    j32 = edge_index[1].astype(jnp.int32)
    pad = _E_PAD - _N_EDGES
    iarr = jnp.pad(i32, (0, pad)).reshape(_NW * _NCH, _SUBS, _G)
    jarr = jnp.pad(j32, (0, pad)).reshape(_NW * _NCH, _SUBS, _G)

    partials = _lr_coulomb_sc(table, iarr, jarr)
    e_mol = jnp.sum(partials.astype(jnp.float64), axis=0)[:_NUM_MOLS]
    return _FACTOR * e_mol


# speedup vs baseline: 667.0933x; 1.1068x over previous
import jax
import jax.numpy as jnp
import numpy as np
from jax import lax
from jax.experimental import pallas as pl
from jax.experimental.pallas import tpu as pltpu
from jax.experimental.pallas import tpu_sc as plsc

_RC = 4.6
_FACTOR = 0.5 * 27.211386245988 * 0.529177210903
_NUM_MOLS = 100
_N_ATOMS = 100000
_N_EDGES = 3200000

_NC = 2
_NS = 16
_NW = _NC * _NS

_B = 2048
_NCH = -(-_N_EDGES // (_NW * _B))
_EPT = _NCH * _B
_E_PAD = _NW * _EPT
_NV = _B // 16


def _compute_chunk(ibufp, jbufp, ri, rj, acc, lanes):
    def _one_vreg(off):
        rsel = lanes + off
        col0 = lanes * 0
        iv = ibufp[pl.ds(off, 16)]
        jv = jbufp[pl.ds(off, 16)]
        xi = plsc.load_gather(ri, [rsel, col0])
        yi = plsc.load_gather(ri, [rsel, col0 + 1])
        zi = plsc.load_gather(ri, [rsel, col0 + 2])
        qi = plsc.load_gather(ri, [rsel, col0 + 3])
        molf = plsc.load_gather(ri, [rsel, col0 + 4])
        xj = plsc.load_gather(rj, [rsel, col0])
        yj = plsc.load_gather(rj, [rsel, col0 + 1])
        zj = plsc.load_gather(rj, [rsel, col0 + 2])
        qj = plsc.load_gather(rj, [rsel, col0 + 3])

        dx = xi - xj
        dy = yi - yj
        dz = zi - zj
        r2 = dx * dx + dy * dy + dz * dz + np.float32(1e-12)
        seed = jnp.int32(0x5F3759DF) - (plsc.bitcast(r2, jnp.int32) >> 1)
        y = plsc.bitcast(seed, jnp.float32)
        for _ in range(3):
            y = y * (np.float32(1.5) - np.float32(0.5) * r2 * y * y)
        inv_d = y
        d = r2 * inv_d
        x = d * np.float32(1.0 / _RC)
        t = jnp.maximum(np.float32(1.0) - x * x, np.float32(1e-6))
        fc = jnp.exp(np.float32(1.0) - np.float32(1.0) / t)
        fc = jnp.where(d < np.float32(_RC), fc, np.float32(0.0))

        mol = molf.astype(jnp.int32)
        e = (np.float32(1.0) - fc) * (qi * qj) * inv_d
        e = jnp.where(iv != jv, e, np.float32(0.0))
        plsc.addupdate_scatter(acc, [lanes, mol], e)

    def vbody(v, carry):
        for u in range(4):
            _one_vreg(v * jnp.int32(64) + jnp.int32(u * 16))
        return carry

    lax.fori_loop(jnp.int32(0), jnp.int32(_NV // 4), vbody, jnp.int32(0),
                  unroll=False)


def _tile_body(table, iarr, jarr, out, ibuf, jbuf, rows_i, rows_j, acc, obuf,
               sem_rows, sem_idx):
    wid = lax.axis_index("s") * _NC + lax.axis_index("c")
    lanes = jnp.arange(16, dtype=jnp.int32)
    zero16 = jnp.zeros(16, dtype=jnp.float32)
    base = wid * jnp.int32(_NCH)

    for r in range(16):
        for cg in range(8):
            acc[r, pl.ds(cg * 16, 16)] = zero16

    def issue_rows(c, p):
        pltpu.async_copy(table.at[ibuf.at[p]], rows_i.at[p], sem_rows)
        pltpu.async_copy(table.at[jbuf.at[p]], rows_j.at[p], sem_rows)

    def wait_rows(p):
        pltpu.make_async_copy(table.at[ibuf.at[p]], rows_i.at[p],
                              sem_rows).wait()
        pltpu.make_async_copy(table.at[jbuf.at[p]], rows_j.at[p],
                              sem_rows).wait()

    def issue_idx(c, p):
        pltpu.async_copy(iarr.at[base + c], ibuf.at[p], sem_idx)
        pltpu.async_copy(jarr.at[base + c], jbuf.at[p], sem_idx)

    def wait_idx(c, p):
        pltpu.make_async_copy(iarr.at[base + c], ibuf.at[p], sem_idx).wait()
        pltpu.make_async_copy(jarr.at[base + c], jbuf.at[p], sem_idx).wait()

    pltpu.sync_copy(iarr.at[base], ibuf.at[jnp.int32(0)])
    pltpu.sync_copy(jarr.at[base], jbuf.at[jnp.int32(0)])
    issue_rows(jnp.int32(0), jnp.int32(0))
    issue_idx(jnp.int32(1), jnp.int32(1))

    def chunk_body(c, carry):
        p = c & jnp.int32(1)
        q = p ^ jnp.int32(1)

        @pl.when(c + jnp.int32(1) < jnp.int32(_NCH))
        def _():
            wait_idx(c + jnp.int32(1), q)
            issue_rows(c + jnp.int32(1), q)

        wait_rows(p)
        _compute_chunk(ibuf.at[p], jbuf.at[p], rows_i.at[p], rows_j.at[p],
                       acc, lanes)

        @pl.when(c + jnp.int32(2) < jnp.int32(_NCH))
        def _():
            issue_idx(c + jnp.int32(2), p)

        return carry

    lax.fori_loop(jnp.int32(0), jnp.int32(_NCH), chunk_body, jnp.int32(0),
                  unroll=False)

    for cg in range(8):
        s = acc[0, pl.ds(cg * 16, 16)]
        for r in range(1, 16):
            s = s + acc[r, pl.ds(cg * 16, 16)]
        obuf[pl.ds(cg * 16, 16)] = s
    pltpu.sync_copy(obuf, out.at[wid])


@jax.jit
def _lr_coulomb_sc(table, iarr, jarr):
    mesh = plsc.VectorSubcoreMesh(core_axis_name="c", subcore_axis_name="s")
    f = pl.kernel(
        _tile_body,
        out_type=jax.ShapeDtypeStruct((_NW, 128), jnp.float32),
        mesh=mesh,
        compiler_params=pltpu.CompilerParams(
            needs_layout_passes=False, use_tc_tiling_on_sc=False),
        scratch_types=[
            pltpu.VMEM((2, _B), jnp.int32),
            pltpu.VMEM((2, _B), jnp.int32),
            pltpu.VMEM((2, _B, 8), jnp.float32),
            pltpu.VMEM((2, _B, 8), jnp.float32),
            pltpu.VMEM((16, 128), jnp.float32),
            pltpu.VMEM((128,), jnp.float32),
            pltpu.SemaphoreType.DMA,
            pltpu.SemaphoreType.DMA,
        ],
    )
    return f(table, iarr, jarr)


def kernel(coord, charges, edge_index, mol_idx):
    coord = coord.astype(jnp.float32)
    q = charges.astype(jnp.float32)
    molf = mol_idx.astype(jnp.float32)
    table = jnp.concatenate(
        [coord, q[:, None], molf[:, None],
         jnp.zeros((_N_ATOMS, 3), jnp.float32)], axis=1)

    i32 = edge_index[0].astype(jnp.int32)
    j32 = edge_index[1].astype(jnp.int32)
    pad = _E_PAD - _N_EDGES
    iarr = jnp.pad(i32, (0, pad)).reshape(_NW * _NCH, _B)
    jarr = jnp.pad(j32, (0, pad)).reshape(_NW * _NCH, _B)

    partials = _lr_coulomb_sc(table, iarr, jarr)
    e_mol = jnp.sum(partials.astype(jnp.float64), axis=0)[:_NUM_MOLS]
    return _FACTOR * e_mol


# speedup vs baseline: 667.4668x; 1.0006x over previous
import jax
import jax.numpy as jnp
import numpy as np
from jax import lax
from jax.experimental import pallas as pl
from jax.experimental.pallas import tpu as pltpu
from jax.experimental.pallas import tpu_sc as plsc

_RC = 4.6
_FACTOR = 0.5 * 27.211386245988 * 0.529177210903
_NUM_MOLS = 100
_N_ATOMS = 100000
_N_EDGES = 3200000

_NC = 2
_NS = 16
_NW = _NC * _NS

_B = 2048
_NCH = -(-_N_EDGES // (_NW * _B))
_EPT = _NCH * _B
_E_PAD = _NW * _EPT
_NV = _B // 16


def _compute_chunk(ibufp, jbufp, ri, rj, acc, lanes):
    def _one_vreg(off):
        rsel = lanes + off
        col0 = lanes * 0
        iv = ibufp[pl.ds(off, 16)]
        jv = jbufp[pl.ds(off, 16)]
        xi = plsc.load_gather(ri, [rsel, col0])
        yi = plsc.load_gather(ri, [rsel, col0 + 1])
        zi = plsc.load_gather(ri, [rsel, col0 + 2])
        qi = plsc.load_gather(ri, [rsel, col0 + 3])
        molf = plsc.load_gather(ri, [rsel, col0 + 4])
        xj = plsc.load_gather(rj, [rsel, col0])
        yj = plsc.load_gather(rj, [rsel, col0 + 1])
        zj = plsc.load_gather(rj, [rsel, col0 + 2])
        qj = plsc.load_gather(rj, [rsel, col0 + 3])

        dx = xi - xj
        dy = yi - yj
        dz = zi - zj
        r2 = dx * dx + dy * dy + dz * dz + np.float32(1e-12)
        seed = jnp.int32(0x5F3759DF) - (plsc.bitcast(r2, jnp.int32) >> 1)
        y = plsc.bitcast(seed, jnp.float32)
        for _ in range(3):
            y = y * (np.float32(1.5) - np.float32(0.5) * r2 * y * y)
        inv_d = y
        d = r2 * inv_d
        x = d * np.float32(1.0 / _RC)
        t = jnp.maximum(np.float32(1.0) - x * x, np.float32(1e-6))
        fc = jnp.exp(np.float32(1.0) - np.float32(1.0) / t)
        fc = jnp.where(d < np.float32(_RC), fc, np.float32(0.0))

        mol = molf.astype(jnp.int32)
        e = (np.float32(1.0) - fc) * (qi * qj) * inv_d
        e = jnp.where(iv != jv, e, np.float32(0.0))
        plsc.addupdate_scatter(acc, [lanes, mol], e)

    def vbody(v, carry):
        for u in range(4):
            _one_vreg(v * jnp.int32(64) + jnp.int32(u * 16))
        return carry

    lax.fori_loop(jnp.int32(0), jnp.int32(_NV // 4), vbody, jnp.int32(0),
                  unroll=False)


def _tile_body(table, iarr, jarr, out, ibuf, jbuf, rows_i, rows_j, acc, obuf,
               sem_rows, sem_idx):
    wid = lax.axis_index("s") * _NC + lax.axis_index("c")
    lanes = jnp.arange(16, dtype=jnp.int32)
    zero16 = jnp.zeros(16, dtype=jnp.float32)
    base = wid * jnp.int32(_NCH)

    for r in range(16):
        for cg in range(8):
            acc[r, pl.ds(cg * 16, 16)] = zero16

    def issue_rows(c, p):
        pltpu.async_copy(table.at[ibuf.at[p]], rows_i.at[p], sem_rows)
        pltpu.async_copy(table.at[jbuf.at[p]], rows_j.at[p], sem_rows)

    def wait_rows(p):
        pltpu.make_async_copy(table.at[ibuf.at[p]], rows_i.at[p],
                              sem_rows).wait()
        pltpu.make_async_copy(table.at[jbuf.at[p]], rows_j.at[p],
                              sem_rows).wait()

    def issue_idx(c, p):
        off = (base + c) * jnp.int32(_B)
        pltpu.async_copy(iarr.at[pl.ds(off, _B)], ibuf.at[p], sem_idx)
        pltpu.async_copy(jarr.at[pl.ds(off, _B)], jbuf.at[p], sem_idx)

    def wait_idx(c, p):
        off = (base + c) * jnp.int32(_B)
        pltpu.make_async_copy(iarr.at[pl.ds(off, _B)], ibuf.at[p],
                              sem_idx).wait()
        pltpu.make_async_copy(jarr.at[pl.ds(off, _B)], jbuf.at[p],
                              sem_idx).wait()

    off0 = base * jnp.int32(_B)
    pltpu.sync_copy(iarr.at[pl.ds(off0, _B)], ibuf.at[jnp.int32(0)])
    pltpu.sync_copy(jarr.at[pl.ds(off0, _B)], jbuf.at[jnp.int32(0)])
    issue_rows(jnp.int32(0), jnp.int32(0))
    issue_idx(jnp.int32(1), jnp.int32(1))

    def chunk_body(c, carry):
        p = c & jnp.int32(1)
        q = p ^ jnp.int32(1)

        @pl.when(c + jnp.int32(1) < jnp.int32(_NCH))
        def _():
            wait_idx(c + jnp.int32(1), q)
            issue_rows(c + jnp.int32(1), q)

        wait_rows(p)
        _compute_chunk(ibuf.at[p], jbuf.at[p], rows_i.at[p], rows_j.at[p],
                       acc, lanes)

        @pl.when(c + jnp.int32(2) < jnp.int32(_NCH))
        def _():
            issue_idx(c + jnp.int32(2), p)

        return carry

    lax.fori_loop(jnp.int32(0), jnp.int32(_NCH), chunk_body, jnp.int32(0),
                  unroll=False)

    for cg in range(8):
        s = acc[0, pl.ds(cg * 16, 16)]
        for r in range(1, 16):
            s = s + acc[r, pl.ds(cg * 16, 16)]
        obuf[pl.ds(cg * 16, 16)] = s
    pltpu.sync_copy(obuf, out.at[wid])


@jax.jit
def _lr_coulomb_sc(table, iarr, jarr):
    mesh = plsc.VectorSubcoreMesh(core_axis_name="c", subcore_axis_name="s")
    f = pl.kernel(
        _tile_body,
        out_type=jax.ShapeDtypeStruct((_NW, 128), jnp.float32),
        mesh=mesh,
        compiler_params=pltpu.CompilerParams(
            needs_layout_passes=False, use_tc_tiling_on_sc=False),
        scratch_types=[
            pltpu.VMEM((2, _B), jnp.int32),
            pltpu.VMEM((2, _B), jnp.int32),
            pltpu.VMEM((2, _B, 8), jnp.float32),
            pltpu.VMEM((2, _B, 8), jnp.float32),
            pltpu.VMEM((16, 128), jnp.float32),
            pltpu.VMEM((128,), jnp.float32),
            pltpu.SemaphoreType.DMA,
            pltpu.SemaphoreType.DMA,
        ],
    )
    return f(table, iarr, jarr)


def kernel(coord, charges, edge_index, mol_idx):
    coord = coord.astype(jnp.float32)
    q = charges.astype(jnp.float32)
    molf = mol_idx.astype(jnp.float32)
    table = jnp.concatenate(
        [coord, q[:, None], molf[:, None],
         jnp.zeros((_N_ATOMS, 3), jnp.float32)], axis=1)

    i32 = edge_index[0].astype(jnp.int32)
    j32 = edge_index[1].astype(jnp.int32)
    pad = _E_PAD - _N_EDGES
    iarr = jnp.pad(i32, (0, pad))
    jarr = jnp.pad(j32, (0, pad))

    partials = _lr_coulomb_sc(table, iarr, jarr)
    e_mol = jnp.sum(partials.astype(jnp.float64), axis=0)[:_NUM_MOLS]
    return _FACTOR * e_mol


# speedup vs baseline: 746.1519x; 1.1179x over previous
import jax
import jax.numpy as jnp
import numpy as np
from jax import lax
from jax.experimental import pallas as pl
from jax.experimental.pallas import tpu as pltpu
from jax.experimental.pallas import tpu_sc as plsc

_RC = 4.6
_FACTOR = 0.5 * 27.211386245988 * 0.529177210903
_NUM_MOLS = 100
_N_ATOMS = 100000
_N_EDGES = 3200000

_NC = 2
_NS = 16
_NW = _NC * _NS

_B = 2000
_EPT = _N_EDGES // _NW
_NCH = _EPT // _B
_NV = _B // 16


def _compute_chunk(ibufp, jbufp, ri, rj, acc, lanes):
    def _one_vreg(off):
        rsel = lanes + off
        col0 = lanes * 0
        iv = ibufp[pl.ds(off, 16)]
        jv = jbufp[pl.ds(off, 16)]
        xi = plsc.load_gather(ri, [rsel, col0])
        yi = plsc.load_gather(ri, [rsel, col0 + 1])
        zi = plsc.load_gather(ri, [rsel, col0 + 2])
        qi = plsc.load_gather(ri, [rsel, col0 + 3])
        molf = plsc.load_gather(ri, [rsel, col0 + 4])
        xj = plsc.load_gather(rj, [rsel, col0])
        yj = plsc.load_gather(rj, [rsel, col0 + 1])
        zj = plsc.load_gather(rj, [rsel, col0 + 2])
        qj = plsc.load_gather(rj, [rsel, col0 + 3])

        dx = xi - xj
        dy = yi - yj
        dz = zi - zj
        r2 = dx * dx + dy * dy + dz * dz + np.float32(1e-12)
        seed = jnp.int32(0x5F3759DF) - (plsc.bitcast(r2, jnp.int32) >> 1)
        y = plsc.bitcast(seed, jnp.float32)
        for _ in range(3):
            y = y * (np.float32(1.5) - np.float32(0.5) * r2 * y * y)
        inv_d = y
        d = r2 * inv_d
        x = d * np.float32(1.0 / _RC)
        t = jnp.maximum(np.float32(1.0) - x * x, np.float32(1e-6))
        fc = jnp.exp(np.float32(1.0) - np.float32(1.0) / t)
        fc = jnp.where(d < np.float32(_RC), fc, np.float32(0.0))

        mol = molf.astype(jnp.int32)
        e = (np.float32(1.0) - fc) * (qi * qj) * inv_d
        e = jnp.where(iv != jv, e, np.float32(0.0))
        plsc.addupdate_scatter(acc, [lanes, mol], e)

    def vbody(v, carry):
        for u in range(5):
            _one_vreg(v * jnp.int32(80) + jnp.int32(u * 16))
        return carry

    lax.fori_loop(jnp.int32(0), jnp.int32(_NV // 5), vbody, jnp.int32(0),
                  unroll=False)


def _tile_body(table, eij, out, ibuf, jbuf, rows_i, rows_j, acc, obuf,
               sem_rows, sem_idx):
    wid = lax.axis_index("s") * _NC + lax.axis_index("c")
    lanes = jnp.arange(16, dtype=jnp.int32)
    zero16 = jnp.zeros(16, dtype=jnp.float32)
    base = wid * jnp.int32(_EPT)

    for r in range(16):
        for cg in range(8):
            acc[r, pl.ds(cg * 16, 16)] = zero16

    def issue_rows(c, p):
        pltpu.async_copy(table.at[ibuf.at[p]], rows_i.at[p], sem_rows)
        pltpu.async_copy(table.at[jbuf.at[p]], rows_j.at[p], sem_rows)

    def wait_rows(p):
        pltpu.make_async_copy(table.at[ibuf.at[p]], rows_i.at[p],
                              sem_rows).wait()
        pltpu.make_async_copy(table.at[jbuf.at[p]], rows_j.at[p],
                              sem_rows).wait()

    def issue_idx(c, p):
        off = base + c * jnp.int32(_B)
        pltpu.async_copy(eij.at[jnp.int32(0), pl.ds(off, _B)], ibuf.at[p],
                         sem_idx)
        pltpu.async_copy(eij.at[jnp.int32(1), pl.ds(off, _B)], jbuf.at[p],
                         sem_idx)

    def wait_idx(c, p):
        off = base + c * jnp.int32(_B)
        pltpu.make_async_copy(eij.at[jnp.int32(0), pl.ds(off, _B)],
                              ibuf.at[p], sem_idx).wait()
        pltpu.make_async_copy(eij.at[jnp.int32(1), pl.ds(off, _B)],
                              jbuf.at[p], sem_idx).wait()

    pltpu.sync_copy(eij.at[jnp.int32(0), pl.ds(base, _B)],
                    ibuf.at[jnp.int32(0)])
    pltpu.sync_copy(eij.at[jnp.int32(1), pl.ds(base, _B)],
                    jbuf.at[jnp.int32(0)])
    issue_rows(jnp.int32(0), jnp.int32(0))
    issue_idx(jnp.int32(1), jnp.int32(1))

    def chunk_body(c, carry):
        p = c & jnp.int32(1)
        q = p ^ jnp.int32(1)

        @pl.when(c + jnp.int32(1) < jnp.int32(_NCH))
        def _():
            wait_idx(c + jnp.int32(1), q)
            issue_rows(c + jnp.int32(1), q)

        wait_rows(p)
        _compute_chunk(ibuf.at[p], jbuf.at[p], rows_i.at[p], rows_j.at[p],
                       acc, lanes)

        @pl.when(c + jnp.int32(2) < jnp.int32(_NCH))
        def _():
            issue_idx(c + jnp.int32(2), p)

        return carry

    lax.fori_loop(jnp.int32(0), jnp.int32(_NCH), chunk_body, jnp.int32(0),
                  unroll=False)

    for cg in range(8):
        s = acc[0, pl.ds(cg * 16, 16)]
        for r in range(1, 16):
            s = s + acc[r, pl.ds(cg * 16, 16)]
        obuf[pl.ds(cg * 16, 16)] = s
    pltpu.sync_copy(obuf, out.at[wid])


@jax.jit
def _lr_coulomb_sc(table, eij):
    mesh = plsc.VectorSubcoreMesh(core_axis_name="c", subcore_axis_name="s")
    f = pl.kernel(
        _tile_body,
        out_type=jax.ShapeDtypeStruct((_NW, 128), jnp.float32),
        mesh=mesh,
        compiler_params=pltpu.CompilerParams(
            needs_layout_passes=False, use_tc_tiling_on_sc=False),
        scratch_types=[
            pltpu.VMEM((2, _B), jnp.int32),
            pltpu.VMEM((2, _B), jnp.int32),
            pltpu.VMEM((2, _B, 8), jnp.float32),
            pltpu.VMEM((2, _B, 8), jnp.float32),
            pltpu.VMEM((16, 128), jnp.float32),
            pltpu.VMEM((128,), jnp.float32),
            pltpu.SemaphoreType.DMA,
            pltpu.SemaphoreType.DMA,
        ],
    )
    return f(table, eij)


def kernel(coord, charges, edge_index, mol_idx):
    coord = coord.astype(jnp.float32)
    q = charges.astype(jnp.float32)
    molf = mol_idx.astype(jnp.float32)
    table = jnp.concatenate(
        [coord, q[:, None], molf[:, None],
         jnp.zeros((_N_ATOMS, 3), jnp.float32)], axis=1)

    eij = edge_index.astype(jnp.int32)

    partials = _lr_coulomb_sc(table, eij)
    e_mol = jnp.sum(partials.astype(jnp.float64), axis=0)[:_NUM_MOLS]
    return _FACTOR * e_mol


# speedup vs baseline: 751.7118x; 1.0075x over previous
import jax
import jax.numpy as jnp
import numpy as np
from jax import lax
from jax.experimental import pallas as pl
from jax.experimental.pallas import tpu as pltpu
from jax.experimental.pallas import tpu_sc as plsc

_RC = 4.6
_FACTOR = 0.5 * 27.211386245988 * 0.529177210903
_NUM_MOLS = 100
_N_ATOMS = 100000
_N_EDGES = 3200000

_NC = 2
_NS = 16
_NW = _NC * _NS

_B = 2000
_EPT = _N_EDGES // _NW
_NCH = _EPT // _B
_NV = _B // 16


def _compute_chunk(ibufp, jbufp, ri, rj, acc, lanes):
    def _one_vreg(off):
        rsel = lanes + off
        col0 = lanes * 0
        iv = ibufp[pl.ds(off, 16)]
        jv = jbufp[pl.ds(off, 16)]
        xi = plsc.load_gather(ri, [rsel, col0])
        yi = plsc.load_gather(ri, [rsel, col0 + 1])
        zi = plsc.load_gather(ri, [rsel, col0 + 2])
        qi = plsc.load_gather(ri, [rsel, col0 + 3])
        molf = plsc.load_gather(ri, [rsel, col0 + 4])
        xj = plsc.load_gather(rj, [rsel, col0])
        yj = plsc.load_gather(rj, [rsel, col0 + 1])
        zj = plsc.load_gather(rj, [rsel, col0 + 2])
        qj = plsc.load_gather(rj, [rsel, col0 + 3])

        dx = xi - xj
        dy = yi - yj
        dz = zi - zj
        r2 = dx * dx + dy * dy + dz * dz + np.float32(1e-12)
        seed = jnp.int32(0x5F3759DF) - (plsc.bitcast(r2, jnp.int32) >> 1)
        y = plsc.bitcast(seed, jnp.float32)
        for _ in range(3):
            y = y * (np.float32(1.5) - np.float32(0.5) * r2 * y * y)
        inv_d = y
        d = r2 * inv_d
        x = d * np.float32(1.0 / _RC)
        t = jnp.maximum(np.float32(1.0) - x * x, np.float32(1e-6))
        fc = jnp.exp(np.float32(1.0) - np.float32(1.0) / t)
        fc = jnp.where(d < np.float32(_RC), fc, np.float32(0.0))

        mol = molf.astype(jnp.int32)
        e = (np.float32(1.0) - fc) * (qi * qj) * inv_d
        e = jnp.where(iv != jv, e, np.float32(0.0))
        plsc.addupdate_scatter(acc, [lanes, mol], e)

    def vbody(v, carry):
        for u in range(5):
            _one_vreg(v * jnp.int32(80) + jnp.int32(u * 16))
        return carry

    lax.fori_loop(jnp.int32(0), jnp.int32(_NV // 5), vbody, jnp.int32(0),
                  unroll=False)


def _tile_body(table, eij, out, shared, ibuf, jbuf, rows_i, rows_j, acc,
               obuf, sem_rows, sem_idx):
    sid = lax.axis_index("s")
    wid = sid * _NC + lax.axis_index("c")
    lanes = jnp.arange(16, dtype=jnp.int32)
    zero16 = jnp.zeros(16, dtype=jnp.float32)
    base = wid * jnp.int32(_EPT)

    for r in range(16):
        for cg in range(8):
            acc[r, pl.ds(cg * 16, 16)] = zero16

    @pl.when(sid == jnp.int32(0))
    def _():
        pltpu.sync_copy(table, shared)

    plsc.subcore_barrier()

    def issue_rows(c, p):
        pltpu.async_copy(shared.at[ibuf.at[p]], rows_i.at[p], sem_rows)
        pltpu.async_copy(shared.at[jbuf.at[p]], rows_j.at[p], sem_rows)

    def wait_rows(p):
        pltpu.make_async_copy(shared.at[ibuf.at[p]], rows_i.at[p],
                              sem_rows).wait()
        pltpu.make_async_copy(shared.at[jbuf.at[p]], rows_j.at[p],
                              sem_rows).wait()

    def issue_idx(c, p):
        off = base + c * jnp.int32(_B)
        pltpu.async_copy(eij.at[jnp.int32(0), pl.ds(off, _B)], ibuf.at[p],
                         sem_idx)
        pltpu.async_copy(eij.at[jnp.int32(1), pl.ds(off, _B)], jbuf.at[p],
                         sem_idx)

    def wait_idx(c, p):
        off = base + c * jnp.int32(_B)
        pltpu.make_async_copy(eij.at[jnp.int32(0), pl.ds(off, _B)],
                              ibuf.at[p], sem_idx).wait()
        pltpu.make_async_copy(eij.at[jnp.int32(1), pl.ds(off, _B)],
                              jbuf.at[p], sem_idx).wait()

    pltpu.sync_copy(eij.at[jnp.int32(0), pl.ds(base, _B)],
                    ibuf.at[jnp.int32(0)])
    pltpu.sync_copy(eij.at[jnp.int32(1), pl.ds(base, _B)],
                    jbuf.at[jnp.int32(0)])
    issue_rows(jnp.int32(0), jnp.int32(0))
    issue_idx(jnp.int32(1), jnp.int32(1))

    def chunk_body(c, carry):
        p = c & jnp.int32(1)
        q = p ^ jnp.int32(1)

        @pl.when(c + jnp.int32(1) < jnp.int32(_NCH))
        def _():
            wait_idx(c + jnp.int32(1), q)
            issue_rows(c + jnp.int32(1), q)

        wait_rows(p)
        _compute_chunk(ibuf.at[p], jbuf.at[p], rows_i.at[p], rows_j.at[p],
                       acc, lanes)

        @pl.when(c + jnp.int32(2) < jnp.int32(_NCH))
        def _():
            issue_idx(c + jnp.int32(2), p)

        return carry

    lax.fori_loop(jnp.int32(0), jnp.int32(_NCH), chunk_body, jnp.int32(0),
                  unroll=False)

    for cg in range(8):
        s = acc[0, pl.ds(cg * 16, 16)]
        for r in range(1, 16):
            s = s + acc[r, pl.ds(cg * 16, 16)]
        obuf[pl.ds(cg * 16, 16)] = s
    pltpu.sync_copy(obuf, out.at[wid])


@jax.jit
def _lr_coulomb_sc(table, eij):
    mesh = plsc.VectorSubcoreMesh(core_axis_name="c", subcore_axis_name="s")
    f = pl.kernel(
        _tile_body,
        out_type=jax.ShapeDtypeStruct((_NW, 128), jnp.float32),
        mesh=mesh,
        compiler_params=pltpu.CompilerParams(
            needs_layout_passes=False, use_tc_tiling_on_sc=False),
        scratch_types=[
            pltpu.VMEM_SHARED((_N_ATOMS, 8), jnp.float32),
            pltpu.VMEM((2, _B), jnp.int32),
            pltpu.VMEM((2, _B), jnp.int32),
            pltpu.VMEM((2, _B, 8), jnp.float32),
            pltpu.VMEM((2, _B, 8), jnp.float32),
            pltpu.VMEM((16, 128), jnp.float32),
            pltpu.VMEM((128,), jnp.float32),
            pltpu.SemaphoreType.DMA,
            pltpu.SemaphoreType.DMA,
        ],
    )
    return f(table, eij)


def kernel(coord, charges, edge_index, mol_idx):
    coord = coord.astype(jnp.float32)
    q = charges.astype(jnp.float32)
    molf = mol_idx.astype(jnp.float32)
    table = jnp.concatenate(
        [coord, q[:, None], molf[:, None],
         jnp.zeros((_N_ATOMS, 3), jnp.float32)], axis=1)

    eij = edge_index.astype(jnp.int32)

    partials = _lr_coulomb_sc(table, eij)
    e_mol = jnp.sum(partials.astype(jnp.float64), axis=0)[:_NUM_MOLS]
    return _FACTOR * e_mol


# speedup vs baseline: 890.7915x; 1.1850x over previous
import jax
import jax.numpy as jnp
import numpy as np
from jax import lax
from jax.experimental import pallas as pl
from jax.experimental.pallas import tpu as pltpu
from jax.experimental.pallas import tpu_sc as plsc

_RC = 4.6
_FACTOR = 0.5 * 27.211386245988 * 0.529177210903
_NUM_MOLS = 100
_N_ATOMS = 100000
_N_EDGES = 3200000

_NC = 2
_NS = 16
_NW = _NC * _NS

_B = 2000
_EPT = _N_EDGES // _NW
_NCH = _EPT // _B
_NV = _B // 16


def _compute_chunk(ibufp, jbufp, ri, rj, acc, lanes, laneoff):
    def _one_vreg(off):
        rsel = lanes + off
        col0 = lanes * 0
        iv = ibufp[pl.ds(off, 16)]
        jv = jbufp[pl.ds(off, 16)]
        xi = plsc.load_gather(ri, [rsel, col0])
        yi = plsc.load_gather(ri, [rsel, col0 + 1])
        zi = plsc.load_gather(ri, [rsel, col0 + 2])
        qi = plsc.load_gather(ri, [rsel, col0 + 3])
        molf = plsc.load_gather(ri, [rsel, col0 + 4])
        xj = plsc.load_gather(rj, [rsel, col0])
        yj = plsc.load_gather(rj, [rsel, col0 + 1])
        zj = plsc.load_gather(rj, [rsel, col0 + 2])
        qj = plsc.load_gather(rj, [rsel, col0 + 3])

        dx = xi - xj
        dy = yi - yj
        dz = zi - zj
        r2 = dx * dx + dy * dy + dz * dz + np.float32(1e-12)
        seed = jnp.int32(0x5F3759DF) - (plsc.bitcast(r2, jnp.int32) >> 1)
        y = plsc.bitcast(seed, jnp.float32)
        hr = np.float32(0.5) * r2
        y = y * (np.float32(1.5) - hr * y * y)
        y = y * (np.float32(1.5) - hr * y * y)
        inv_d = y
        t = jnp.maximum(np.float32(1.0) - r2 * np.float32(1.0 / (_RC * _RC)),
                        np.float32(1e-6))
        fc = jnp.exp(np.float32(1.0) - np.float32(1.0) / t)

        mol = molf.astype(jnp.int32)
        e = (np.float32(1.0) - fc) * (qi * qj) * inv_d
        e = jnp.where(iv != jv, e, np.float32(0.0))
        plsc.addupdate_scatter(acc, [laneoff + mol], e)

    def vbody(v, carry):
        for u in range(25):
            _one_vreg(v * jnp.int32(400) + jnp.int32(u * 16))
        return carry

    lax.fori_loop(jnp.int32(0), jnp.int32(_NV // 25), vbody, jnp.int32(0),
                  unroll=False)


def _tile_body(table, eij, out, shared, ibuf, jbuf, rows_i, rows_j, acc,
               obuf, sem_rows, sem_idx):
    sid = lax.axis_index("s")
    wid = sid * _NC + lax.axis_index("c")
    lanes = jnp.arange(16, dtype=jnp.int32)
    laneoff = lanes * jnp.int32(128)
    zero16 = jnp.zeros(16, dtype=jnp.float32)
    base = wid * jnp.int32(_EPT)

    for w in range(128):
        acc[pl.ds(w * 16, 16)] = zero16

    @pl.when(sid == jnp.int32(0))
    def _():
        pltpu.sync_copy(table, shared)

    plsc.subcore_barrier()

    def issue_rows(c, p):
        pltpu.async_copy(shared.at[ibuf.at[p]], rows_i.at[p], sem_rows)
        pltpu.async_copy(shared.at[jbuf.at[p]], rows_j.at[p], sem_rows)

    def wait_rows(p):
        pltpu.make_async_copy(shared.at[ibuf.at[p]], rows_i.at[p],
                              sem_rows).wait()
        pltpu.make_async_copy(shared.at[jbuf.at[p]], rows_j.at[p],
                              sem_rows).wait()

    def issue_idx(c, p):
        off = base + c * jnp.int32(_B)
        pltpu.async_copy(eij.at[jnp.int32(0), pl.ds(off, _B)], ibuf.at[p],
                         sem_idx)
        pltpu.async_copy(eij.at[jnp.int32(1), pl.ds(off, _B)], jbuf.at[p],
                         sem_idx)

    def wait_idx(c, p):
        off = base + c * jnp.int32(_B)
        pltpu.make_async_copy(eij.at[jnp.int32(0), pl.ds(off, _B)],
                              ibuf.at[p], sem_idx).wait()
        pltpu.make_async_copy(eij.at[jnp.int32(1), pl.ds(off, _B)],
                              jbuf.at[p], sem_idx).wait()

    pltpu.sync_copy(eij.at[jnp.int32(0), pl.ds(base, _B)],
                    ibuf.at[jnp.int32(0)])
    pltpu.sync_copy(eij.at[jnp.int32(1), pl.ds(base, _B)],
                    jbuf.at[jnp.int32(0)])
    issue_rows(jnp.int32(0), jnp.int32(0))
    issue_idx(jnp.int32(1), jnp.int32(1))

    def chunk_body(c, carry):
        p = c & jnp.int32(1)
        q = p ^ jnp.int32(1)

        @pl.when(c + jnp.int32(1) < jnp.int32(_NCH))
        def _():
            wait_idx(c + jnp.int32(1), q)
            issue_rows(c + jnp.int32(1), q)

        wait_rows(p)
        _compute_chunk(ibuf.at[p], jbuf.at[p], rows_i.at[p], rows_j.at[p],
                       acc, lanes, laneoff)

        @pl.when(c + jnp.int32(2) < jnp.int32(_NCH))
        def _():
            issue_idx(c + jnp.int32(2), p)

        return carry

    lax.fori_loop(jnp.int32(0), jnp.int32(_NCH), chunk_body, jnp.int32(0),
                  unroll=False)

    for cg in range(8):
        s = acc[pl.ds(cg * 16, 16)]
        for r in range(1, 16):
            s = s + acc[pl.ds(r * 128 + cg * 16, 16)]
        obuf[pl.ds(cg * 16, 16)] = s
    pltpu.sync_copy(obuf, out.at[wid])


@jax.jit
def _lr_coulomb_sc(table, eij):
    mesh = plsc.VectorSubcoreMesh(core_axis_name="c", subcore_axis_name="s")
    f = pl.kernel(
        _tile_body,
        out_type=jax.ShapeDtypeStruct((_NW, 128), jnp.float32),
        mesh=mesh,
        compiler_params=pltpu.CompilerParams(
            needs_layout_passes=False, use_tc_tiling_on_sc=False),
        scratch_types=[
            pltpu.VMEM_SHARED((_N_ATOMS, 8), jnp.float32),
            pltpu.VMEM((2, _B), jnp.int32),
            pltpu.VMEM((2, _B), jnp.int32),
            pltpu.VMEM((2, _B, 8), jnp.float32),
            pltpu.VMEM((2, _B, 8), jnp.float32),
            pltpu.VMEM((2048,), jnp.float32),
            pltpu.VMEM((128,), jnp.float32),
            pltpu.SemaphoreType.DMA,
            pltpu.SemaphoreType.DMA,
        ],
    )
    return f(table, eij)


def kernel(coord, charges, edge_index, mol_idx):
    coord = coord.astype(jnp.float32)
    q = charges.astype(jnp.float32)
    molf = mol_idx.astype(jnp.float32)
    table = jnp.concatenate(
        [coord, q[:, None], molf[:, None],
         jnp.zeros((_N_ATOMS, 3), jnp.float32)], axis=1)

    eij = edge_index.astype(jnp.int32)

    partials = _lr_coulomb_sc(table, eij)
    e_mol = jnp.sum(partials.astype(jnp.float64), axis=0)[:_NUM_MOLS]
    return _FACTOR * e_mol
